# Initial kernel scaffold; baseline (speedup 1.0000x reference)
import jax
import jax.numpy as jnp
from jax.experimental import pallas as pl


def kernel(pids, ctx, class_ctx, prefix_emb, suffix_emb, sos_emb, eos_emb):
    raise NotImplementedError("write your pallas kernel here")


# speedup vs baseline: 1.1968x; 1.1968x over previous
import functools

import jax
import jax.numpy as jnp
from jax import lax
from jax.experimental import pallas as pl
from jax.experimental.pallas import tpu as pltpu
from jax.experimental.pallas import tpu_sc as plsc

L = 16
NC = 2
NS = 16
NW = NC * NS


def _build(B, D, n_ctx, pre_len, suf_len, V, interpret=False):
    ROW = n_ctx * D
    SEQ = 1 + pre_len + n_ctx + suf_len + 1
    OUT_ROW = SEQ * D
    COMB_OFF = (1 + pre_len) * D
    BPW = B // NW
    CHUNK = 8
    NCHUNK = BPW // CHUNK

    mesh = plsc.VectorSubcoreMesh(core_axis_name="c", subcore_axis_name="s")

    @functools.partial(
        pl.kernel,
        out_type=jax.ShapeDtypeStruct((B, OUT_ROW), jnp.float32),
        mesh=mesh,
        interpret=interpret,
        scratch_types=[
            pltpu.VMEM((BPW,), jnp.int32),
            pltpu.VMEM((ROW,), jnp.float32),
            pltpu.VMEM((CHUNK, ROW), jnp.float32),
            pltpu.VMEM((CHUNK, OUT_ROW), jnp.float32),
            pltpu.SemaphoreType.DMA,
        ],
    )
    def k(pids_hbm, ctx_hbm, table_hbm, pre_hbm, suf_hbm, sos_hbm, eos_hbm,
          out_hbm, idx_v, ctx_v, gath_v, outbuf, sem):
        wid = lax.axis_index("s") * NC + lax.axis_index("c")
        base = wid * BPW
        pltpu.sync_copy(pids_hbm.at[pl.ds(base, BPW)], idx_v)
        pltpu.sync_copy(ctx_hbm, ctx_v)
        for r in range(CHUNK):
            pltpu.sync_copy(sos_hbm, outbuf.at[r, pl.ds(0, D)])
            pltpu.sync_copy(pre_hbm, outbuf.at[r, pl.ds(D, pre_len * D)])
            pltpu.sync_copy(suf_hbm, outbuf.at[r, pl.ds(COMB_OFF + ROW, suf_len * D)])
            pltpu.sync_copy(eos_hbm, outbuf.at[r, pl.ds(COMB_OFF + ROW + suf_len * D, D)])

        def chunk_body(c, _):
            pltpu.async_copy(
                table_hbm.at[idx_v.at[pl.ds(c * CHUNK, CHUNK)]], gath_v, sem
            ).wait()
            for r in range(CHUNK):
                @pl.loop(0, ROW // L)
                def _add(j):
                    s = j * L
                    outbuf[r, pl.ds(COMB_OFF + s, L)] = (
                        gath_v[r, pl.ds(s, L)] + ctx_v[pl.ds(s, L)]
                    )
            pltpu.sync_copy(outbuf, out_hbm.at[pl.ds(base + c * CHUNK, CHUNK)])
            return ()

        lax.fori_loop(0, NCHUNK, chunk_body, (), unroll=False)

    return k


def kernel(pids, ctx, class_ctx, prefix_emb, suffix_emb, sos_emb, eos_emb):
    B = pids.shape[0]
    n_ctx, D = ctx.shape
    V = class_ctx.shape[0]
    pre_len = prefix_emb.shape[0]
    suf_len = suffix_emb.shape[0]
    SEQ = 1 + pre_len + n_ctx + suf_len + 1

    k = _build(B, D, n_ctx, pre_len, suf_len, V)
    out = k(
        pids.astype(jnp.int32),
        ctx.reshape(-1),
        class_ctx.reshape(V, n_ctx * D),
        prefix_emb.reshape(-1),
        suffix_emb.reshape(-1),
        sos_emb.reshape(-1),
        eos_emb.reshape(-1),
    )
    return out.reshape(B, SEQ, D)


# speedup vs baseline: 5.7439x; 4.7993x over previous
import functools

import jax
import jax.numpy as jnp
from jax import lax
from jax.experimental import pallas as pl
from jax.experimental.pallas import tpu as pltpu
from jax.experimental.pallas import tpu_sc as plsc

L = 16
NC = 2
NS = 16
NW = NC * NS


def _build(B, D, n_ctx, pre_len, suf_len, V):
    SEQ = 1 + pre_len + n_ctx + suf_len + 1
    COMB = 1 + pre_len
    BPW = B // NW
    CHUNK = 8
    NCHUNK = BPW // CHUNK

    mesh = plsc.VectorSubcoreMesh(core_axis_name="c", subcore_axis_name="s")

    @functools.partial(
        pl.kernel,
        out_type=jax.ShapeDtypeStruct((SEQ, B, D), jnp.float32),
        mesh=mesh,
        scratch_types=[
            pltpu.VMEM((BPW,), jnp.int32),
            pltpu.VMEM((n_ctx, D), jnp.float32),
            pltpu.VMEM((CHUNK, n_ctx, D), jnp.float32),
            pltpu.VMEM((SEQ, CHUNK, D), jnp.float32),
            pltpu.SemaphoreType.DMA,
        ],
    )
    def k(pids_hbm, ctx_hbm, table_hbm, pre_hbm, suf_hbm, sos_hbm, eos_hbm,
          out_hbm, idx_v, ctx_v, gath_v, outbuf, sem):
        wid = lax.axis_index("s") * NC + lax.axis_index("c")
        base = wid * BPW
        pltpu.sync_copy(pids_hbm.at[pl.ds(base, BPW)], idx_v)
        pltpu.sync_copy(ctx_hbm, ctx_v)
        for r in range(CHUNK):
            pltpu.sync_copy(sos_hbm.at[0], outbuf.at[0, r])
            for p in range(pre_len):
                pltpu.sync_copy(pre_hbm.at[p], outbuf.at[1 + p, r])
            for p in range(suf_len):
                pltpu.sync_copy(suf_hbm.at[p], outbuf.at[COMB + n_ctx + p, r])
            pltpu.sync_copy(eos_hbm.at[0], outbuf.at[SEQ - 1, r])

        def chunk_body(c, _):
            pltpu.async_copy(
                table_hbm.at[idx_v.at[pl.ds(c * CHUNK, CHUNK)]], gath_v, sem
            ).wait()
            for r in range(CHUNK):
                for cc in range(n_ctx):
                    @pl.loop(0, D // L)
                    def _add(j):
                        s = j * L
                        outbuf[COMB + cc, r, pl.ds(s, L)] = (
                            gath_v[r, cc, pl.ds(s, L)] + ctx_v[cc, pl.ds(s, L)]
                        )
            pltpu.sync_copy(outbuf, out_hbm.at[:, pl.ds(base + c * CHUNK, CHUNK), :])
            return ()

        lax.fori_loop(0, NCHUNK, chunk_body, (), unroll=False)

    return k


def kernel(pids, ctx, class_ctx, prefix_emb, suffix_emb, sos_emb, eos_emb):
    B = pids.shape[0]
    n_ctx, D = ctx.shape
    V = class_ctx.shape[0]
    pre_len = prefix_emb.shape[0]
    suf_len = suffix_emb.shape[0]

    k = _build(B, D, n_ctx, pre_len, suf_len, V)
    out = k(pids.astype(jnp.int32), ctx, class_ctx, prefix_emb, suffix_emb,
            sos_emb, eos_emb)
    return out.transpose(1, 0, 2)


# speedup vs baseline: 9.6885x; 1.6867x over previous
import functools

import jax
import jax.numpy as jnp
from jax import lax
from jax.experimental import pallas as pl
from jax.experimental.pallas import tpu as pltpu
from jax.experimental.pallas import tpu_sc as plsc

L = 16
NC = 2
NS = 16
NW = NC * NS


def _build(B, D, n_ctx, pre_len, suf_len, V):
    SEQ = 1 + pre_len + n_ctx + suf_len + 1
    COMB = 1 + pre_len
    HEAD = COMB
    TAIL = suf_len + 1
    BPW = B // NW
    CHUNK = 8
    NCHUNK = BPW // CHUNK

    mesh = plsc.VectorSubcoreMesh(core_axis_name="c", subcore_axis_name="s")

    @functools.partial(
        pl.kernel,
        out_type=jax.ShapeDtypeStruct((SEQ, B, D), jnp.float32),
        mesh=mesh,
        scratch_types=[
            pltpu.VMEM((BPW,), jnp.int32),
            pltpu.VMEM((n_ctx, D), jnp.float32),
            pltpu.VMEM((HEAD + TAIL, D), jnp.float32),
            pltpu.VMEM((CHUNK, n_ctx, D), jnp.float32),
            pltpu.VMEM((CHUNK, n_ctx, D), jnp.float32),
            pltpu.VMEM((n_ctx, CHUNK, D), jnp.float32),
            pltpu.VMEM((n_ctx, CHUNK, D), jnp.float32),
            pltpu.VMEM((HEAD, CHUNK, D), jnp.float32),
            pltpu.VMEM((TAIL, CHUNK, D), jnp.float32),
            pltpu.SemaphoreType.DMA,
            pltpu.SemaphoreType.DMA,
            pltpu.SemaphoreType.DMA,
            pltpu.SemaphoreType.DMA,
        ],
    )
    def k(pids_hbm, ctx_hbm, table_hbm, pre_hbm, suf_hbm, sos_hbm, eos_hbm,
          out_hbm, idx_v, ctx_v, const_v, gath0, gath1, comb0, comb1,
          head_v, tail_v, sg0, sg1, so0, so1):
        gath = (gath0, gath1)
        comb = (comb0, comb1)
        sg = (sg0, sg1)
        so = (so0, so1)
        wid = lax.axis_index("s") * NC + lax.axis_index("c")
        base = wid * BPW
        pltpu.sync_copy(pids_hbm.at[pl.ds(base, BPW)], idx_v)
        pltpu.sync_copy(ctx_hbm, ctx_v)
        pltpu.sync_copy(sos_hbm, const_v.at[pl.ds(0, 1)])
        pltpu.sync_copy(pre_hbm, const_v.at[pl.ds(1, pre_len)])
        pltpu.sync_copy(suf_hbm, const_v.at[pl.ds(HEAD, suf_len)])
        pltpu.sync_copy(eos_hbm, const_v.at[pl.ds(HEAD + suf_len, 1)])

        @pl.loop(0, HEAD)
        def _fh(p):
            @pl.loop(0, D // L)
            def _fj(j):
                s = j * L
                row = const_v[p, pl.ds(s, L)]

                @pl.loop(0, CHUNK, unroll=CHUNK)
                def _fr(r):
                    head_v[p, r, pl.ds(s, L)] = row

        @pl.loop(0, TAIL)
        def _ft(p):
            @pl.loop(0, D // L)
            def _fj(j):
                s = j * L
                row = const_v[HEAD + p, pl.ds(s, L)]

                @pl.loop(0, CHUNK, unroll=CHUNK)
                def _fr(r):
                    tail_v[p, r, pl.ds(s, L)] = row

        def start_gather(c):
            b = c % 2
            pltpu.async_copy(
                table_hbm.at[idx_v.at[pl.ds(c * CHUNK, CHUNK)]],
                gath[b], sg[b])

        def wait_gather(c):
            b = c % 2
            pltpu.make_async_copy(
                table_hbm.at[pl.ds(0, CHUNK)], gath[b], sg[b]).wait()

        def start_out(c):
            b = c % 2
            rows = pl.ds(base + c * CHUNK, CHUNK)
            pltpu.async_copy(comb[b],
                             out_hbm.at[pl.ds(COMB, n_ctx), rows, :], so[b])
            pltpu.async_copy(head_v, out_hbm.at[pl.ds(0, HEAD), rows, :], so[b])
            pltpu.async_copy(tail_v,
                             out_hbm.at[pl.ds(COMB + n_ctx, TAIL), rows, :],
                             so[b])

        def wait_out(c):
            b = c % 2
            rows = pl.ds(0, CHUNK)
            pltpu.make_async_copy(
                comb[b], out_hbm.at[pl.ds(COMB, n_ctx), rows, :],
                so[b]).wait()
            pltpu.make_async_copy(
                head_v, out_hbm.at[pl.ds(0, HEAD), rows, :], so[b]).wait()
            pltpu.make_async_copy(
                tail_v, out_hbm.at[pl.ds(COMB + n_ctx, TAIL), rows, :],
                so[b]).wait()

        def compute(c):
            b = c % 2
            gb = gath[b]
            cb = comb[b]

            @pl.loop(0, n_ctx)
            def _cc(cc):
                @pl.loop(0, D // L)
                def _cj(j):
                    s = j * L
                    cvec = ctx_v[cc, pl.ds(s, L)]

                    @pl.loop(0, CHUNK, unroll=CHUNK)
                    def _cr(r):
                        cb[cc, r, pl.ds(s, L)] = gb[r, cc, pl.ds(s, L)] + cvec

        start_gather(0)
        start_gather(1)
        for c in range(NCHUNK):
            wait_gather(c)
            if c >= 2:
                wait_out(c - 2)
            compute(c)
            start_out(c)
            if c + 2 < NCHUNK:
                start_gather(c + 2)
        wait_out(NCHUNK - 2)
        wait_out(NCHUNK - 1)

    return k


def kernel(pids, ctx, class_ctx, prefix_emb, suffix_emb, sos_emb, eos_emb):
    B = pids.shape[0]
    n_ctx, D = ctx.shape
    V = class_ctx.shape[0]
    pre_len = prefix_emb.shape[0]
    suf_len = suffix_emb.shape[0]

    k = _build(B, D, n_ctx, pre_len, suf_len, V)
    out = k(pids.astype(jnp.int32), ctx, class_ctx, prefix_emb, suffix_emb,
            sos_emb, eos_emb)
    return out.transpose(1, 0, 2)
